# Initial kernel scaffold; baseline (speedup 1.0000x reference)
import functools

import jax
import jax.numpy as jnp
import numpy as np
from jax.experimental import pallas as pl

_ANCHORS = np.array(
    [[1.3221 / 13.0, 1.73145 / 13.0],
     [3.19275 / 13.0, 4.00944 / 13.0],
     [5.05587 / 13.0, 8.09892 / 13.0],
     [9.47112 / 13.0, 4.84053 / 13.0],
     [11.2364 / 13.0, 10.0071 / 13.0]], dtype=np.float32)
_THRESH = 0.6
_PRIOR_ITER = 12800

_A = 5
_K = 50


def _per_truth(x1, y1, x2, y2, w_grid, h_grid):
    cw = x2 - x1
    ch = y2 - y1
    a1 = cw * ch
    best_iou = jnp.zeros_like(cw)
    best_ind = jnp.zeros_like(cw)
    for a in range(_A):
        aw = float(_ANCHORS[a, 0])
        ah = float(_ANCHORS[a, 1])
        a2 = float(np.float32(_ANCHORS[a, 0]) * np.float32(_ANCHORS[a, 1]))
        inter = jnp.minimum(cw, aw) * jnp.minimum(ch, ah)
        union = jnp.clip(a1 + a2 - inter, 1e-12, None)
        iou = inter / union
        upd = iou > best_iou
        best_ind = jnp.where(upd, float(a), best_ind)
        best_iou = jnp.where(upd, iou, best_iou)
    dx = (x1 + x2) / 2.0 * w_grid
    dy = (y1 + y2) / 2.0 * h_grid
    gxk = jnp.ceil(dx) - 1.0
    gyk = jnp.ceil(dy) - 1.0
    pos = gyk * w_grid + gxk
    return pos, best_ind, best_iou, cw, ch, dx - gxk, dy - gyk


def _loss_body(x_ref, tj_ref, tk_ref, out_ref, *, grid_h, grid_w):
    hw = grid_h * grid_w
    x = x_ref[0]
    tj = tj_ref[0]
    tk = tk_ref[0]

    posk, indk, miouk, cwk, chk, fxk, fyk = _per_truth(
        tk[0:1, :], tk[1:2, :], tk[2:3, :], tk[3:4, :],
        float(grid_w), float(grid_h))
    validk = miouk != 0.0
    rdx = -jnp.log(1.0 / fxk - 1.0)
    rdy = -jnp.log(1.0 / fyk - 1.0)
    aw_sel = jnp.zeros_like(indk)
    ah_sel = jnp.zeros_like(indk)
    for a in range(_A):
        hit = indk == float(a)
        aw_sel = jnp.where(hit, float(_ANCHORS[a, 0]), aw_sel)
        ah_sel = jnp.where(hit, float(_ANCHORS[a, 1]), ah_sel)
    t2t = jnp.log(cwk) / aw_sel
    t3t = jnp.log(chk) / ah_sel
    cf2 = 2.0 - cwk * chk

    posj, indj, miouj = _per_truth(
        tj[:, 0:1], tj[:, 1:2], tj[:, 2:3], tj[:, 3:4],
        float(grid_w), float(grid_h))[:3]

    iota_k = jax.lax.broadcasted_iota(jnp.float32, (1, _K), 1)
    iota_j = jax.lax.broadcasted_iota(jnp.float32, (_K, 1), 0)
    same = (posj == posk) & (indj == indk)
    beats = same & ((miouj > miouk) | ((miouj == miouk) & (iota_j < iota_k)))
    winner = validk & jnp.logical_not(jnp.any(beats, axis=0, keepdims=True))

    cell = jax.lax.broadcasted_iota(jnp.int32, (hw, 1), 0)
    gx = (cell % grid_w).astype(jnp.float32)
    gy = (cell // grid_w).astype(jnp.float32)
    cellf = cell.astype(jnp.float32)

    tx1 = tk[0:1, :]
    ty1 = tk[1:2, :]
    tx2 = tk[2:3, :]
    ty2 = tk[3:4, :]
    a2 = (tx2 - tx1) * (ty2 - ty1)

    acc_prior = jnp.float32(0.0)
    acc_noobj = jnp.float32(0.0)
    acc_coord = jnp.float32(0.0)
    for a in range(_A):
        base = a * 25
        t0 = x[:, base + 0:base + 1]
        t1 = x[:, base + 1:base + 2]
        t2 = x[:, base + 2:base + 3]
        t3 = x[:, base + 3:base + 4]
        t4 = x[:, base + 4:base + 5]
        aw = float(_ANCHORS[a, 0])
        ah = float(_ANCHORS[a, 1])
        c0 = (1.0 / (1.0 + jnp.exp(-t0)) + gx) / float(grid_w)
        c1 = (1.0 / (1.0 + jnp.exp(-t1)) + gy) / float(grid_h)
        wa = jnp.exp(t2) * aw
        ha = jnp.exp(t3) * ah
        bx1 = c0 - wa / 2.0
        bx2 = c0 + wa / 2.0
        by1 = c1 - ha / 2.0
        by2 = c1 + ha / 2.0
        a1 = (bx2 - bx1) * (by2 - by1)
        acc_prior += jnp.sum((wa - aw) ** 2) + jnp.sum((ha - ah) ** 2)
        ix = jnp.clip(jnp.minimum(bx2, tx2) - jnp.maximum(bx1, tx1), 0.0, None)
        iy = jnp.clip(jnp.minimum(by2, ty2) - jnp.maximum(by1, ty1), 0.0, None)
        inter = ix * iy
        union = jnp.clip(a1 + a2 - inter, 1e-12, None)
        iou = inter / union
        maxiou = jnp.max(iou, axis=1, keepdims=True)
        noobj = maxiou < _THRESH
        match = (posk == cellf) & (indk == float(a)) & validk
        anymatch = jnp.any(match, axis=1, keepdims=True)
        acc_noobj += jnp.sum(
            jnp.where(noobj & jnp.logical_not(anymatch), t4, 0.0) ** 2)
        wm = match & winner
        d0 = t0 - rdx
        d1 = t1 - rdy
        d2 = t2 - t2t
        d3 = t3 - t3t
        sq = d0 * d0 + d1 * d1 + d2 * d2 + d3 * d3
        acc_coord += jnp.sum(jnp.where(wm, cf2 * sq, 0.0))

    out_ref[0] = jnp.concatenate(
        [acc_prior.reshape(1, 1), acc_noobj.reshape(1, 1),
         acc_coord.reshape(1, 1)], axis=1)


def kernel(output, truths, iteration):
    b, grid_h, grid_w, t = output.shape
    hw = grid_h * grid_w
    x = output.reshape(b, hw, t)
    tt = jnp.transpose(truths, (0, 2, 1))
    body = functools.partial(_loss_body, grid_h=grid_h, grid_w=grid_w)
    parts = pl.pallas_call(
        body,
        grid=(b,),
        in_specs=[
            pl.BlockSpec((1, hw, t), lambda i: (i, 0, 0)),
            pl.BlockSpec((1, _K, 5), lambda i: (i, 0, 0)),
            pl.BlockSpec((1, 5, _K), lambda i: (i, 0, 0)),
        ],
        out_specs=pl.BlockSpec((1, 1, 3), lambda i: (i, 0, 0)),
        out_shape=jax.ShapeDtypeStruct((b, 1, 3), jnp.float32),
    )(x, truths, tt)
    sums = jnp.sum(parts, axis=(0, 1))
    prior = jnp.where(iteration < _PRIOR_ITER, sums[0], jnp.float32(0.0))
    return prior + sums[1] + sums[2]


# speedup vs baseline: 20.4391x; 20.4391x over previous
import functools

import jax
import jax.numpy as jnp
import numpy as np
from jax.experimental import pallas as pl

_ANCHORS = np.array(
    [[1.3221 / 13.0, 1.73145 / 13.0],
     [3.19275 / 13.0, 4.00944 / 13.0],
     [5.05587 / 13.0, 8.09892 / 13.0],
     [9.47112 / 13.0, 4.84053 / 13.0],
     [11.2364 / 13.0, 10.0071 / 13.0]], dtype=np.float32)
_THRESH = 0.6
_PRIOR_ITER = 12800

_A = 5
_K = 50


def _per_truth(x1, y1, x2, y2, w_grid, h_grid):
    cw = x2 - x1
    ch = y2 - y1
    a1 = cw * ch
    best_iou = jnp.zeros_like(cw)
    best_ind = jnp.zeros_like(cw)
    for a in range(_A):
        aw = float(_ANCHORS[a, 0])
        ah = float(_ANCHORS[a, 1])
        a2 = float(np.float32(_ANCHORS[a, 0]) * np.float32(_ANCHORS[a, 1]))
        inter = jnp.minimum(cw, aw) * jnp.minimum(ch, ah)
        union = jnp.clip(a1 + a2 - inter, 1e-12, None)
        iou = inter / union
        upd = iou > best_iou
        best_ind = jnp.where(upd, float(a), best_ind)
        best_iou = jnp.where(upd, iou, best_iou)
    dx = (x1 + x2) / 2.0 * w_grid
    dy = (y1 + y2) / 2.0 * h_grid
    gxk = jnp.ceil(dx) - 1.0
    gyk = jnp.ceil(dy) - 1.0
    pos = gyk * w_grid + gxk
    return pos, best_ind, best_iou, cw, ch, dx - gxk, dy - gyk


def _loss_body(x_ref, tj_ref, tk_ref, out_ref, *, grid_h, grid_w):
    hw = grid_h * grid_w
    x = x_ref[0]
    tj = tj_ref[0]
    tk = tk_ref[0]

    posk, indk, miouk, cwk, chk, fxk, fyk = _per_truth(
        tk[0:1, :], tk[1:2, :], tk[2:3, :], tk[3:4, :],
        float(grid_w), float(grid_h))
    validk = miouk != 0.0
    rdx = -jnp.log(1.0 / fxk - 1.0)
    rdy = -jnp.log(1.0 / fyk - 1.0)
    aw_sel = jnp.zeros_like(indk)
    ah_sel = jnp.zeros_like(indk)
    for a in range(_A):
        hit = indk == float(a)
        aw_sel = jnp.where(hit, float(_ANCHORS[a, 0]), aw_sel)
        ah_sel = jnp.where(hit, float(_ANCHORS[a, 1]), ah_sel)
    t2t = jnp.log(cwk) / aw_sel
    t3t = jnp.log(chk) / ah_sel
    cf2 = 2.0 - cwk * chk

    posj, indj, miouj = _per_truth(
        tj[:, 0:1], tj[:, 1:2], tj[:, 2:3], tj[:, 3:4],
        float(grid_w), float(grid_h))[:3]

    iota_k = jax.lax.broadcasted_iota(jnp.int32, (1, _K), 1).astype(jnp.float32)
    iota_j = jax.lax.broadcasted_iota(jnp.int32, (_K, 1), 0).astype(jnp.float32)
    same = (posj == posk) & (indj == indk)
    beats = same & ((miouj > miouk) | ((miouj == miouk) & (iota_j < iota_k)))
    winner = validk & jnp.logical_not(jnp.any(beats, axis=0, keepdims=True))

    cell = jax.lax.broadcasted_iota(jnp.int32, (hw, 1), 0)
    gx = (cell % grid_w).astype(jnp.float32)
    gy = (cell // grid_w).astype(jnp.float32)
    cellf = cell.astype(jnp.float32)

    tx1 = tk[0:1, :]
    ty1 = tk[1:2, :]
    tx2 = tk[2:3, :]
    ty2 = tk[3:4, :]
    a2 = (tx2 - tx1) * (ty2 - ty1)

    acc_prior = jnp.float32(0.0)
    acc_noobj = jnp.float32(0.0)
    acc_coord = jnp.float32(0.0)
    for a in range(_A):
        base = a * 25
        t0 = x[:, base + 0:base + 1]
        t1 = x[:, base + 1:base + 2]
        t2 = x[:, base + 2:base + 3]
        t3 = x[:, base + 3:base + 4]
        t4 = x[:, base + 4:base + 5]
        aw = float(_ANCHORS[a, 0])
        ah = float(_ANCHORS[a, 1])
        c0 = (1.0 / (1.0 + jnp.exp(-t0)) + gx) / float(grid_w)
        c1 = (1.0 / (1.0 + jnp.exp(-t1)) + gy) / float(grid_h)
        wa = jnp.exp(t2) * aw
        ha = jnp.exp(t3) * ah
        bx1 = c0 - wa / 2.0
        bx2 = c0 + wa / 2.0
        by1 = c1 - ha / 2.0
        by2 = c1 + ha / 2.0
        a1 = (bx2 - bx1) * (by2 - by1)
        acc_prior += jnp.sum((wa - aw) ** 2) + jnp.sum((ha - ah) ** 2)
        ix = jnp.clip(jnp.minimum(bx2, tx2) - jnp.maximum(bx1, tx1), 0.0, None)
        iy = jnp.clip(jnp.minimum(by2, ty2) - jnp.maximum(by1, ty1), 0.0, None)
        inter = ix * iy
        union = jnp.clip(a1 + a2 - inter, 1e-12, None)
        iou = inter / union
        maxiou = jnp.max(iou, axis=1, keepdims=True)
        noobj = maxiou < _THRESH
        match = (posk == cellf) & (indk == float(a)) & validk
        anymatch = jnp.any(match, axis=1, keepdims=True)
        acc_noobj += jnp.sum(
            jnp.where(noobj & jnp.logical_not(anymatch), t4, 0.0) ** 2)
        wm = match & winner
        d0 = t0 - rdx
        d1 = t1 - rdy
        d2 = t2 - t2t
        d3 = t3 - t3t
        sq = d0 * d0 + d1 * d1 + d2 * d2 + d3 * d3
        acc_coord += jnp.sum(jnp.where(wm, cf2 * sq, 0.0))

    out_ref[0] = jnp.concatenate(
        [acc_prior.reshape(1, 1), acc_noobj.reshape(1, 1),
         acc_coord.reshape(1, 1)], axis=1)


def kernel(output, truths, iteration):
    b, grid_h, grid_w, t = output.shape
    hw = grid_h * grid_w
    x = output.reshape(b, hw, t)
    tt = jnp.transpose(truths, (0, 2, 1))
    body = functools.partial(_loss_body, grid_h=grid_h, grid_w=grid_w)
    parts = pl.pallas_call(
        body,
        grid=(b,),
        in_specs=[
            pl.BlockSpec((1, hw, t), lambda i: (i, 0, 0)),
            pl.BlockSpec((1, _K, 5), lambda i: (i, 0, 0)),
            pl.BlockSpec((1, 5, _K), lambda i: (i, 0, 0)),
        ],
        out_specs=pl.BlockSpec((1, 1, 3), lambda i: (i, 0, 0)),
        out_shape=jax.ShapeDtypeStruct((b, 1, 3), jnp.float32),
    )(x, truths, tt)
    sums = jnp.sum(parts, axis=(0, 1))
    prior = jnp.where(iteration < _PRIOR_ITER, sums[0], jnp.float32(0.0))
    return prior + sums[1] + sums[2]


# speedup vs baseline: 58.1857x; 2.8468x over previous
import functools

import jax
import jax.numpy as jnp
import numpy as np
from jax.experimental import pallas as pl

_ANCHORS = np.array(
    [[1.3221 / 13.0, 1.73145 / 13.0],
     [3.19275 / 13.0, 4.00944 / 13.0],
     [5.05587 / 13.0, 8.09892 / 13.0],
     [9.47112 / 13.0, 4.84053 / 13.0],
     [11.2364 / 13.0, 10.0071 / 13.0]], dtype=np.float32)
_THRESH = 0.6
_PRIOR_ITER = 12800

_A = 5
_K = 50


def _per_truth(x1, y1, x2, y2, w_grid, h_grid):
    cw = x2 - x1
    ch = y2 - y1
    a1 = cw * ch
    best_iou = jnp.zeros_like(cw)
    best_ind = jnp.zeros_like(cw)
    for a in range(_A):
        aw = float(_ANCHORS[a, 0])
        ah = float(_ANCHORS[a, 1])
        a2 = float(np.float32(_ANCHORS[a, 0]) * np.float32(_ANCHORS[a, 1]))
        inter = jnp.minimum(cw, aw) * jnp.minimum(ch, ah)
        union = jnp.clip(a1 + a2 - inter, 1e-12, None)
        iou = inter / union
        upd = iou > best_iou
        best_ind = jnp.where(upd, float(a), best_ind)
        best_iou = jnp.where(upd, iou, best_iou)
    dx = (x1 + x2) / 2.0 * w_grid
    dy = (y1 + y2) / 2.0 * h_grid
    gxk = jnp.ceil(dx) - 1.0
    gyk = jnp.ceil(dy) - 1.0
    pos = gyk * w_grid + gxk
    return pos, best_ind, best_iou, cw, ch, dx - gxk, dy - gyk


def _loss_body(x_ref, tj_ref, tk_ref, out_ref, *, grid_h, grid_w):
    hw = grid_h * grid_w
    x = x_ref[0]
    tj = tj_ref[0]
    tk = tk_ref[0]

    posj, indj, miouj, cwj, chj, fxj, fyj = _per_truth(
        tj[:, 0:1], tj[:, 1:2], tj[:, 2:3], tj[:, 3:4],
        float(grid_w), float(grid_h))
    validj = miouj != 0.0
    rdx = -jnp.log(1.0 / fxj - 1.0)
    rdy = -jnp.log(1.0 / fyj - 1.0)
    aw_sel = jnp.zeros_like(indj)
    ah_sel = jnp.zeros_like(indj)
    for a in range(_A):
        hit = indj == float(a)
        aw_sel = jnp.where(hit, float(_ANCHORS[a, 0]), aw_sel)
        ah_sel = jnp.where(hit, float(_ANCHORS[a, 1]), ah_sel)
    t2t = jnp.log(cwj) / aw_sel
    t3t = jnp.log(chj) / ah_sel
    cf2 = 2.0 - cwj * chj

    posm, indm, mioum = _per_truth(
        tk[0:1, :], tk[1:2, :], tk[2:3, :], tk[3:4, :],
        float(grid_w), float(grid_h))[:3]

    iota_j = jax.lax.broadcasted_iota(jnp.int32, (_K, 1), 0).astype(jnp.float32)
    iota_m = jax.lax.broadcasted_iota(jnp.int32, (1, _K), 1).astype(jnp.float32)
    same = (posj == posm) & (indj == indm)
    beats = same & ((mioum > miouj) | ((mioum == miouj) & (iota_m < iota_j)))
    winner = validj & jnp.logical_not(jnp.any(beats, axis=1, keepdims=True))

    cell = jax.lax.broadcasted_iota(jnp.int32, (1, hw), 1)
    gx = (cell % grid_w).astype(jnp.float32)
    gy = (cell // grid_w).astype(jnp.float32)
    cellf = cell.astype(jnp.float32)

    tx1 = tj[:, 0:1]
    ty1 = tj[:, 1:2]
    tx2 = tj[:, 2:3]
    ty2 = tj[:, 3:4]
    a2 = (tx2 - tx1) * (ty2 - ty1)

    acc_prior = jnp.float32(0.0)
    acc_noobj = jnp.float32(0.0)
    acc_coord = jnp.float32(0.0)
    for a in range(_A):
        base = a * 5
        t0 = x[base + 0:base + 1, :]
        t1 = x[base + 1:base + 2, :]
        t2 = x[base + 2:base + 3, :]
        t3 = x[base + 3:base + 4, :]
        t4 = x[base + 4:base + 5, :]
        aw = float(_ANCHORS[a, 0])
        ah = float(_ANCHORS[a, 1])
        c0 = (1.0 / (1.0 + jnp.exp(-t0)) + gx) / float(grid_w)
        c1 = (1.0 / (1.0 + jnp.exp(-t1)) + gy) / float(grid_h)
        wa = jnp.exp(t2) * aw
        ha = jnp.exp(t3) * ah
        bx1 = c0 - wa / 2.0
        bx2 = c0 + wa / 2.0
        by1 = c1 - ha / 2.0
        by2 = c1 + ha / 2.0
        a1 = (bx2 - bx1) * (by2 - by1)
        acc_prior += jnp.sum((wa - aw) ** 2) + jnp.sum((ha - ah) ** 2)
        ix = jnp.clip(jnp.minimum(bx2, tx2) - jnp.maximum(bx1, tx1), 0.0, None)
        iy = jnp.clip(jnp.minimum(by2, ty2) - jnp.maximum(by1, ty1), 0.0, None)
        inter = ix * iy
        union = jnp.clip(a1 + a2 - inter, 1e-12, None)
        iou = inter / union
        maxiou = jnp.max(iou, axis=0, keepdims=True)
        noobj = maxiou < _THRESH
        match = (posj == cellf) & (indj == float(a)) & validj
        anymatch = jnp.any(match, axis=0, keepdims=True)
        acc_noobj += jnp.sum(
            jnp.where(noobj & jnp.logical_not(anymatch), t4, 0.0) ** 2)
        wm = match & winner
        d0 = t0 - rdx
        d1 = t1 - rdy
        d2 = t2 - t2t
        d3 = t3 - t3t
        sq = d0 * d0 + d1 * d1 + d2 * d2 + d3 * d3
        acc_coord += jnp.sum(jnp.where(wm, cf2 * sq, 0.0))

    out_ref[0] = jnp.concatenate(
        [acc_prior.reshape(1, 1), acc_noobj.reshape(1, 1),
         acc_coord.reshape(1, 1)], axis=1)


def kernel(output, truths, iteration):
    b, grid_h, grid_w, t = output.shape
    hw = grid_h * grid_w
    xt = output.reshape(b, hw, _A, t // _A)[:, :, :, 0:5]
    xt = jnp.transpose(xt, (0, 2, 3, 1)).reshape(b, _A * 5, hw)
    tt = jnp.transpose(truths, (0, 2, 1))
    body = functools.partial(_loss_body, grid_h=grid_h, grid_w=grid_w)
    parts = pl.pallas_call(
        body,
        grid=(b,),
        in_specs=[
            pl.BlockSpec((1, _A * 5, hw), lambda i: (i, 0, 0)),
            pl.BlockSpec((1, _K, 5), lambda i: (i, 0, 0)),
            pl.BlockSpec((1, 5, _K), lambda i: (i, 0, 0)),
        ],
        out_specs=pl.BlockSpec((1, 1, 3), lambda i: (i, 0, 0)),
        out_shape=jax.ShapeDtypeStruct((b, 1, 3), jnp.float32),
    )(xt, truths, tt)
    sums = jnp.sum(parts, axis=(0, 1))
    prior = jnp.where(iteration < _PRIOR_ITER, sums[0], jnp.float32(0.0))
    return prior + sums[1] + sums[2]
